# baseline (speedup 1.0000x reference)
import functools
import math

import jax
import jax.numpy as jnp
import numpy as np
from jax.experimental import pallas as pl
from jax.experimental.pallas import tpu as pltpu

C = 96
IDIM = 64
PATCH = 4
HEADS = 8
NPROMPT = 4
B, H, W = 2, 224, 224
HP, WP = H // PATCH, W // PATCH
N = HP * WP
KSEL = 784
HID = 192

F32 = jnp.float32


def _gelu(x):
    return 0.5 * x * (1.0 + jax.lax.erf(x * (2.0 ** -0.5)))


def _dot(a, b):
    return jnp.dot(a, b, preferred_element_type=F32)


def _dg(a, b, dims):
    return jax.lax.dot_general(a, b, (dims, ((), ())),
                               preferred_element_type=F32)


def _dft_mats():
    j = np.arange(W, dtype=np.int64)
    ang = -2.0 * np.pi * ((j[:, None] * j[None, :]) % W).astype(np.float64) / W
    return (np.cos(ang).astype(np.float32), np.sin(ang).astype(np.float32))


def _pool_mats():
    pm = np.zeros((HP, W), np.float32)
    for i in range(HP):
        pm[i, 4 * i:4 * i + 4] = 0.25
    return pm, pm.T.copy()


FTOP = 120


def _sym_pool_mats():
    pm, _ = _pool_mats()
    pma = np.zeros((HP, FTOP), np.float32)
    pma[:, :113] = pm[:, :113]
    pmb = np.zeros((HP, FTOP), np.float32)
    for rp in range(1, 112):
        pmb[:, rp] = pm[:, 224 - rp]
    pmtf = np.zeros((W, HP), np.float32)
    for c in range(W):
        pmtf[c, :] = pm[:, (224 - c) % 224] * 1.0
    return pma, pmb, pmtf.copy()


def _expand_mats():
    e = np.zeros((H, HP), np.float32)
    for i in range(H):
        e[i, i // 4] = 1.0
    return e, e.T.copy()


def _pos_feat():
    yy, xx = np.meshgrid(np.linspace(-1.0, 1.0, HP),
                         np.linspace(-1.0, 1.0, WP), indexing='ij')
    rr = np.sqrt(xx ** 2 + yy ** 2)
    ang = np.arctan2(yy, xx) / math.pi
    return np.stack([rr, ang], axis=-1).reshape(1, N, 2).astype(np.float32)


def _spectral_body(xv_ref, xi_ref, dr_ref, di_ref, drt_ref, dit_ref,
                   pm_ref, pmt_ref, pma_ref, pmb_ref, pmtf_ref,
                   av_ref, ai_ref, cv_ref, ci_ref, sv_ref, si_ref,
                   pv_ref, pi_ref, mv_ref, mi_ref):
    dr = dr_ref[...]
    di = di_ref[...]
    drt = drt_ref[...]
    dit = dit_ref[...]
    pm = pm_ref[...]
    pmt = pmt_ref[...]
    pma = pma_ref[...]
    pmb = pmb_ref[...]
    pmtf = pmtf_ref[...]

    def one(x, a_ref, c_ref, s_ref, xp_ref, xm_ref):
        yr = _dot(x, dr)
        yi = _dot(x, di)
        fr = _dot(dr, yr) - _dot(di, yi)
        fi = _dot(dr, yi) + _dot(di, yr)
        m2 = fr * fr + fi * fi
        mag = jnp.sqrt(m2)
        amp = jnp.log1p(mag)
        rinv = 1.0 / jnp.maximum(mag, 1e-30)
        cosp = fr * rinv
        sinp = fi * rinv
        a_ref[0] = _dot(_dot(pm, amp), pmt)
        c_ref[0] = _dot(_dot(pm, cosp), pmt)
        s_ref[0] = _dot(_dot(pm, sinp), pmt)
        xp_ref[0] = _dot(_dot(pm, x), pmt)
        xm_ref[0] = jnp.mean(x, axis=1, keepdims=True).mean(axis=0,
                                                           keepdims=True)

    one(xv_ref[0], av_ref, cv_ref, sv_ref, pv_ref, mv_ref)
    one(xi_ref[0], ai_ref, ci_ref, si_ref, pi_ref, mi_ref)


def _spectral(vis_s, ir_s, dr, di, drt, dit, pm, pmt, pma, pmb, pmtf):
    nimg = vis_s.shape[0]
    pool_s = jax.ShapeDtypeStruct((nimg, HP, WP), F32)
    mean_s = jax.ShapeDtypeStruct((nimg, 1, 1), F32)
    small = pl.BlockSpec((1, HP, WP), lambda g: (g, 0, 0))
    tiny = pl.BlockSpec((1, 1, 1), lambda g: (g, 0, 0))
    full = lambda s: pl.BlockSpec(s, lambda g: tuple(0 for _ in s))
    return pl.pallas_call(
        _spectral_body,
        grid=(nimg,),
        in_specs=[
            pl.BlockSpec((1, H, W), lambda g: (g, 0, 0)),
            pl.BlockSpec((1, H, W), lambda g: (g, 0, 0)),
            full((H, W)), full((H, W)),
            full((FTOP, W)), full((FTOP, W)),
            full((HP, W)), full((W, HP)),
            full((HP, FTOP)), full((HP, FTOP)), full((W, HP)),
        ],
        out_specs=(small, small, small, small, small, small, small, small,
                   tiny, tiny),
        out_shape=(pool_s,) * 8 + (mean_s,) * 2,
    )(vis_s, ir_s, dr, di, drt, dit, pm, pmt, pma, pmb, pmtf)


def _intent_body(pooled_ref, rw1t_ref, rb1_ref, rw2t_ref, rb2_ref,
                 iat_ref, iab_ref, ipt_ref, ipb_ref, a_ref, p_ref):
    pooled = pooled_ref[...]
    h = _gelu(_dot(pooled, rw1t_ref[...]) + rb1_ref[...])
    lg = _dot(h, rw2t_ref[...]) + rb2_ref[...]
    m = jnp.max(lg, axis=-1, keepdims=True)
    e = jnp.exp(lg - m)
    intent = e / jnp.sum(e, axis=-1, keepdims=True)
    a_ref[...] = _dot(intent, iat_ref[...]) + iab_ref[...]
    p_ref[...] = _dot(intent, ipt_ref[...]) + ipb_ref[...]


def _intent(pooled, p):
    rw1t = p['rW1'].T
    rw2t = p['rW2'].T
    iat = p['iaW'][:, :NPROMPT].T
    ipt = p['ipW'][:, :NPROMPT].T
    full = lambda s: pl.BlockSpec(s, lambda: tuple(0 for _ in s))
    return pl.pallas_call(
        _intent_body,
        in_specs=[full((B, 2 * C)), full((2 * C, 128)), full((1, 128)),
                  full((128, NPROMPT)), full((1, NPROMPT)),
                  full((NPROMPT, IDIM)), full((1, IDIM)),
                  full((NPROMPT, IDIM)), full((1, IDIM))],
        out_specs=(full((B, IDIM)), full((B, IDIM))),
        out_shape=(jax.ShapeDtypeStruct((B, IDIM), F32),
                   jax.ShapeDtypeStruct((B, IDIM), F32)),
    )(pooled, rw1t, p['rb1'][None], rw2t, p['rb2'][None],
      iat, p['iab'][None], ipt, p['ipb'][None])


def _score_body(x_ref, w1_ref, b1_ref, w2_ref, b2_ref, w3_ref, b3_ref, o_ref):
    h = _gelu(_dot(x_ref[...], w1_ref[...]) + b1_ref[...])
    h = _gelu(_dot(h, w2_ref[...]) + b2_ref[...])
    o_ref[...] = _dot(h, w3_ref[...]) + b3_ref[...]


def _scores(feat, w1, b1, w2, b2, w3, b3):
    rows, fdim = feat.shape
    blk = 784
    grid = rows // blk
    full = lambda s: pl.BlockSpec(s, lambda g: tuple(0 for _ in s))
    out = pl.pallas_call(
        _score_body,
        grid=(grid,),
        in_specs=[pl.BlockSpec((blk, fdim), lambda g: (g, 0)),
                  full((fdim, HID)), full((1, HID)),
                  full((HID, HID)), full((1, HID)),
                  full((HID, 1)), full((1, 1))],
        out_specs=pl.BlockSpec((blk, 1), lambda g: (g, 0)),
        out_shape=jax.ShapeDtypeStruct((rows, 1), F32),
    )(feat, w1.T, b1[None], w2.T, b2[None], w3.T, b3[None])
    return out


def _select_body(k, srow_ref, scol_ref, rep_ref, ctx_ref, selc_ref, selr_ref):
    srow = srow_ref[0]
    nchunk = 8
    cb = N // nchunk
    for ib in range(nchunk):
        si = scol_ref[0, ib * cb:(ib + 1) * cb, :]
        gt = jnp.sum((srow > si).astype(F32), axis=1, keepdims=True)
        jm = jax.lax.broadcasted_iota(jnp.int32, (cb, N), 1)
        im = jax.lax.broadcasted_iota(jnp.int32, (cb, N), 0) + ib * cb
        eq = jnp.sum(((srow == si) & (jm < im)).astype(F32),
                     axis=1, keepdims=True)
        sel = ((gt + eq) < k).astype(F32)
        selc_ref[ib * cb:(ib + 1) * cb, :] = sel
        selr_ref[:, ib * cb:(ib + 1) * cb] = sel.T
    selr = selr_ref[...]
    for ib in range(nchunk):
        jm = jax.lax.broadcasted_iota(jnp.int32, (cb, N), 1)
        im = jax.lax.broadcasted_iota(jnp.int32, (cb, N), 0) + ib * cb
        pos = jnp.sum(selr * (jm < im).astype(F32), axis=1, keepdims=True)
        sel = selc_ref[ib * cb:(ib + 1) * cb, :]
        ri = jax.lax.broadcasted_iota(jnp.int32, (cb, k), 1)
        oh = ((pos.astype(jnp.int32) == ri) & (sel > 0.5)).astype(F32)
        contrib = _dg(oh, rep_ref[0, ib * cb:(ib + 1) * cb, :],
                      ((0,), (0,)))
        if ib == 0:
            ctx_ref[0] = contrib
        else:
            ctx_ref[0] += contrib


def _select_gather(scores, rep, k):
    b, n, d = rep.shape
    srow = scores.reshape(b, 1, n)
    scol = scores.reshape(b, n, 1)
    return pl.pallas_call(
        functools.partial(_select_body, k),
        grid=(b,),
        in_specs=[pl.BlockSpec((1, 1, n), lambda g: (g, 0, 0)),
                  pl.BlockSpec((1, n, 1), lambda g: (g, 0, 0)),
                  pl.BlockSpec((1, n, d), lambda g: (g, 0, 0))],
        out_specs=pl.BlockSpec((1, k, d), lambda g: (g, 0, 0)),
        out_shape=jax.ShapeDtypeStruct((b, k, d), F32),
        scratch_shapes=[pltpu.VMEM((n, 1), F32),
                        pltpu.VMEM((1, n), F32)],
    )(srow, scol, rep)


def _attn_body(d, q_ref, ctx_ref, n1g_ref, n1b_ref, wqt_ref, bq_ref,
               wkt_ref, bk_ref, wvt_ref, bv_ref, wot_ref, bo_ref,
               n2g_ref, n2b_ref, w1t_ref, b1_ref, w2t_ref, b2_ref, o_ref):
    hd = d // HEADS
    scale = hd ** -0.5
    qin = q_ref[0]
    m = jnp.mean(qin, axis=-1, keepdims=True)
    v = jnp.mean((qin - m) ** 2, axis=-1, keepdims=True)
    qn = (qin - m) / jnp.sqrt(v + 1e-5) * n1g_ref[...] + n1b_ref[...]
    q = _dot(qn, wqt_ref[...]) + bq_ref[...]
    c = ctx_ref[0]
    kk = _dot(c, wkt_ref[...]) + bk_ref[...]
    vv = _dot(c, wvt_ref[...]) + bv_ref[...]
    outs = []
    for h in range(HEADS):
        sl = slice(h * hd, (h + 1) * hd)
        lg = _dg(q[:, sl], kk[:, sl], ((1,), (1,))) * scale
        mx = jnp.max(lg, axis=-1, keepdims=True)
        e = jnp.exp(lg - mx)
        a = e / jnp.sum(e, axis=-1, keepdims=True)
        outs.append(_dot(a, vv[:, sl]))
    o = jnp.concatenate(outs, axis=1)
    o = qin + _dot(o, wot_ref[...]) + bo_ref[...]
    m2 = jnp.mean(o, axis=-1, keepdims=True)
    v2 = jnp.mean((o - m2) ** 2, axis=-1, keepdims=True)
    o2 = (o - m2) / jnp.sqrt(v2 + 1e-5) * n2g_ref[...] + n2b_ref[...]
    hm = _gelu(_dot(o2, w1t_ref[...]) + b1_ref[...])
    o_ref[0] = o + _dot(hm, w2t_ref[...]) + b2_ref[...]


def _cross_attn(q_in, ctx, p, pre):
    b, n, d = q_in.shape
    k = ctx.shape[1]
    qb = 392
    grid = (b, n // qb)
    wfull = lambda s: pl.BlockSpec(s, lambda gb, gi: tuple(0 for _ in s))
    args = [q_in, ctx,
            p[pre + 'n1g'][None], p[pre + 'n1b'][None],
            p[pre + 'Wq'].T, p[pre + 'bq'][None],
            p[pre + 'Wk'].T, p[pre + 'bk'][None],
            p[pre + 'Wv'].T, p[pre + 'bv'][None],
            p[pre + 'Wo'].T, p[pre + 'bo'][None],
            p[pre + 'n2g'][None], p[pre + 'n2b'][None],
            p[pre + 'W1'].T, p[pre + 'b1'][None],
            p[pre + 'W2'].T, p[pre + 'b2'][None]]
    in_specs = [pl.BlockSpec((1, qb, d), lambda gb, gi: (gb, gi, 0)),
                pl.BlockSpec((1, k, d), lambda gb, gi: (gb, 0, 0)),
                wfull((1, d)), wfull((1, d)),
                wfull((d, d)), wfull((1, d)),
                wfull((d, d)), wfull((1, d)),
                wfull((d, d)), wfull((1, d)),
                wfull((d, d)), wfull((1, d)),
                wfull((1, d)), wfull((1, d)),
                wfull((d, 2 * d)), wfull((1, 2 * d)),
                wfull((2 * d, d)), wfull((1, d))]
    return pl.pallas_call(
        functools.partial(_attn_body, d),
        grid=grid,
        in_specs=in_specs,
        out_specs=pl.BlockSpec((1, qb, d), lambda gb, gi: (gb, gi, 0)),
        out_shape=jax.ShapeDtypeStruct((b, n, d), F32),
    )(*args)


def _upbase_body(f_ref, v_ref, i_ref, e_ref, et_ref, o_ref):
    up = _dot(e_ref[...], _dot(f_ref[0], et_ref[...]))
    o_ref[0] = 0.5 * (v_ref[0] + i_ref[0]) + up


def _upbase(fmap, vis, ir, e, et):
    nimg = fmap.shape[0]
    return pl.pallas_call(
        _upbase_body,
        grid=(nimg,),
        in_specs=[pl.BlockSpec((1, HP, WP), lambda g: (g, 0, 0)),
                  pl.BlockSpec((1, H, W), lambda g: (g, 0, 0)),
                  pl.BlockSpec((1, H, W), lambda g: (g, 0, 0)),
                  pl.BlockSpec((H, HP), lambda g: (0, 0)),
                  pl.BlockSpec((HP, W), lambda g: (0, 0))],
        out_specs=pl.BlockSpec((1, H, W), lambda g: (g, 0, 0)),
        out_shape=jax.ShapeDtypeStruct((nimg, H, W), F32),
    )(fmap, vis, ir, e, et)


def _chan_mm_body(act, w_ref, b_ref, x_ref, o_ref):
    r = _dg(w_ref[...], x_ref[0], ((1,), (0,))) + b_ref[...]
    o_ref[0] = _gelu(r) if act else r


def _chan_matmul(x, wmat, bias, act):
    b, cin, cols = x.shape
    cout = wmat.shape[0]
    blk = cols // 8
    return pl.pallas_call(
        functools.partial(_chan_mm_body, act),
        grid=(b, 8),
        in_specs=[pl.BlockSpec((cout, cin), lambda gb, gi: (0, 0)),
                  pl.BlockSpec((cout, 1), lambda gb, gi: (0, 0)),
                  pl.BlockSpec((1, cin, blk), lambda gb, gi: (gb, 0, gi))],
        out_specs=pl.BlockSpec((1, cout, blk), lambda gb, gi: (gb, 0, gi)),
        out_shape=jax.ShapeDtypeStruct((b, cout, cols), F32),
    )(wmat, bias[:, None], x)


_STRIP = 28 * W


def _conv3_body(w_ref, b_ref, m0_ref, m2_ref, top_ref, bot_ref,
                xc_ref, o_ref):
    w = w_ref[...]
    z8 = jnp.zeros((C, 8), F32)
    xx = jnp.concatenate([z8, top_ref[0, 0], xc_ref[0], bot_ref[0, 0], z8],
                         axis=1)
    acc = None
    for dy in range(3):
        for dx in range(3):
            s = 7 + dy * W + dx
            xs = xx[:, s:s + _STRIP]
            g = dy * 3 + dx
            t = _dg(w[:, g * C:(g + 1) * C], xs, ((1,), (0,)))
            if dx == 0:
                t = t * m0_ref[...]
            elif dx == 2:
                t = t * m2_ref[...]
            acc = t if acc is None else acc + t
    o_ref[0] = acc + b_ref[...]


def _conv3(h1_flat, w2, bias, m0, m2, tops, bots):
    full = lambda s: pl.BlockSpec(s, lambda gb, gi: tuple(0 for _ in s))
    return pl.pallas_call(
        _conv3_body,
        grid=(B, 8),
        in_specs=[full((C, 9 * C)), full((C, 1)),
                  full((1, _STRIP)), full((1, _STRIP)),
                  pl.BlockSpec((1, 1, C, W), lambda gb, gi: (gb, gi, 0, 0)),
                  pl.BlockSpec((1, 1, C, W), lambda gb, gi: (gb, gi, 0, 0)),
                  pl.BlockSpec((1, C, _STRIP),
                               lambda gb, gi: (gb, 0, gi))],
        out_specs=pl.BlockSpec((1, C, _STRIP), lambda gb, gi: (gb, 0, gi)),
        out_shape=jax.ShapeDtypeStruct((B, C, H * W), F32),
    )(w2, bias[:, None], m0, m2, tops, bots, h1_flat)


def kernel(vis, ir, params):
    p = params
    dr, di = _dft_mats()
    pm, pmt = _pool_mats()
    pma, pmb, pmtf = _sym_pool_mats()
    e, et = _expand_mats()
    drt = jnp.asarray(dr[:FTOP].copy()); dit = jnp.asarray(di[:FTOP].copy())
    dr = jnp.asarray(dr); di = jnp.asarray(di)
    pm = jnp.asarray(pm); pmt = jnp.asarray(pmt)
    pma = jnp.asarray(pma); pmb = jnp.asarray(pmb); pmtf = jnp.asarray(pmtf)
    e = jnp.asarray(e); et = jnp.asarray(et)
    pos = jnp.asarray(_pos_feat())

    (av_p, ai_p, cv_p, ci_p, sv_p, si_p, pv_p, pi_p, mv, mi) = _spectral(
        vis.reshape(B * C, H, W), ir.reshape(B * C, H, W), dr, di, drt, dit,
        pm, pmt, pma, pmb, pmtf)

    def tok(arr3):
        return arr3.reshape(B, C, N).transpose(0, 2, 1)

    av_t, ai_t = tok(av_p), tok(ai_p)
    cv_t, ci_t = tok(cv_p), tok(ci_p)
    sv_t, si_t = tok(sv_p), tok(si_p)
    vis_t, ir_t = tok(pv_p), tok(pi_p)

    pooled = jnp.concatenate([mv.reshape(B, C), mi.reshape(B, C)], axis=1)
    amp_intent, phase_intent = _intent(pooled, p)

    pos_b = jnp.broadcast_to(pos, (B, N, 2))
    ai_b = jnp.broadcast_to(amp_intent[:, None, :], (B, N, IDIM))
    pi_b = jnp.broadcast_to(phase_intent[:, None, :], (B, N, IDIM))
    amp_feat = jnp.concatenate([av_t, ai_t, vis_t, ir_t, pos_b, ai_b],
                               axis=-1).reshape(B * N, -1)
    ph_feat = jnp.concatenate([cv_t, sv_t, ci_t, si_t, av_t, ai_t,
                               vis_t, ir_t, pos_b, pi_b],
                              axis=-1).reshape(B * N, -1)
    amp_scores = _scores(amp_feat, p['aW1'], p['ab1'], p['aW2'], p['ab2'],
                         p['aW3'], p['ab3']).reshape(B, N)
    ph_scores = _scores(ph_feat, p['pW1'], p['pb1'], p['pW2'], p['pb2'],
                        p['pW3'], p['pb3']).reshape(B, N)

    amp_rep = 0.5 * (av_t + ai_t)
    ph_rep = jnp.concatenate([0.5 * (cv_t + ci_t), 0.5 * (sv_t + si_t)],
                             axis=-1)
    amp_ctx = _select_gather(amp_scores, amp_rep, KSEL)
    ph_ctx = _select_gather(ph_scores, ph_rep, KSEL)

    q_amp = 0.5 * (vis_t + ir_t)
    q_ph = jnp.concatenate([vis_t, ir_t], axis=-1)
    amp_out = _cross_attn(q_amp, amp_ctx, p, 'ac_')
    ph_out = _cross_attn(q_ph, ph_ctx, p, 'pc_')
    ph_red = 0.5 * (ph_out[..., :C] + ph_out[..., C:])
    fused = amp_out + ph_red

    fmap = fused.transpose(0, 2, 1).reshape(B * C, HP, WP)
    base = _upbase(fmap, vis.reshape(B * C, H, W), ir.reshape(B * C, H, W),
                   e, et)
    base = base.reshape(B, C, H * W)
    h1 = _chan_matmul(base, p['c1W'], p['c1b'], act=True)

    w2 = p['c2W'].transpose(0, 2, 3, 1).reshape(C, 9 * C)
    lane = np.arange(_STRIP) % W
    m0 = jnp.asarray((lane != 0).astype(np.float32)[None])
    m2 = jnp.asarray((lane != W - 1).astype(np.float32)[None])
    h1r = h1.reshape(B, C, H, W)
    zrow = jnp.zeros((B, C, 1, W), F32)
    tops = jnp.concatenate([zrow, h1r[:, :, 27:196:28, :]],
                           axis=2).transpose(0, 2, 1, 3)
    bots = jnp.concatenate([h1r[:, :, 28:224:28, :], zrow],
                           axis=2).transpose(0, 2, 1, 3)
    out = _conv3(h1, w2, p['c2b'], m0, m2, tops, bots)
    return out.reshape(B, C, H, W)


# speedup vs baseline: 1.0952x; 1.0952x over previous
import functools
import math

import jax
import jax.numpy as jnp
import numpy as np
from jax.experimental import pallas as pl
from jax.experimental.pallas import tpu as pltpu

C = 96
IDIM = 64
PATCH = 4
HEADS = 8
NPROMPT = 4
B, H, W = 2, 224, 224
HP, WP = H // PATCH, W // PATCH
N = HP * WP
KSEL = 784
HID = 192

F32 = jnp.float32


def _gelu(x):
    return 0.5 * x * (1.0 + jax.lax.erf(x * (2.0 ** -0.5)))


def _dot(a, b):
    return jnp.dot(a, b, preferred_element_type=F32)


def _dg(a, b, dims):
    return jax.lax.dot_general(a, b, (dims, ((), ())),
                               preferred_element_type=F32)


def _dft_mats():
    j = np.arange(W, dtype=np.int64)
    ang = -2.0 * np.pi * ((j[:, None] * j[None, :]) % W).astype(np.float64) / W
    return (np.cos(ang).astype(np.float32), np.sin(ang).astype(np.float32))


def _pool_mats():
    pm = np.zeros((HP, W), np.float32)
    for i in range(HP):
        pm[i, 4 * i:4 * i + 4] = 0.25
    return pm, pm.T.copy()


FTOP = 120


def _sym_pool_mats():
    pm, _ = _pool_mats()
    pma = np.zeros((HP, FTOP), np.float32)
    pma[:, :113] = pm[:, :113]
    pmb = np.zeros((HP, FTOP), np.float32)
    for rp in range(1, 112):
        pmb[:, rp] = pm[:, 224 - rp]
    pmtf = np.zeros((W, HP), np.float32)
    for c in range(W):
        pmtf[c, :] = pm[:, (224 - c) % 224] * 1.0
    return pma, pmb, pmtf.copy()


def _expand_mats():
    e = np.zeros((H, HP), np.float32)
    for i in range(H):
        e[i, i // 4] = 1.0
    return e, e.T.copy()


def _pos_feat():
    yy, xx = np.meshgrid(np.linspace(-1.0, 1.0, HP),
                         np.linspace(-1.0, 1.0, WP), indexing='ij')
    rr = np.sqrt(xx ** 2 + yy ** 2)
    ang = np.arctan2(yy, xx) / math.pi
    return np.stack([rr, ang], axis=-1).reshape(1, N, 2).astype(np.float32)


def _spectral_body(xv_ref, xi_ref, dr_ref, di_ref, drt_ref, dit_ref,
                   pm_ref, pmt_ref, pma_ref, pmb_ref, pmtf_ref,
                   av_ref, ai_ref, cv_ref, ci_ref, sv_ref, si_ref,
                   pv_ref, pi_ref, mv_ref, mi_ref):
    dr = dr_ref[...]
    di = di_ref[...]
    drt = drt_ref[...]
    dit = dit_ref[...]
    pm = pm_ref[...]
    pmt = pmt_ref[...]
    pma = pma_ref[...]
    pmb = pmb_ref[...]
    pmtf = pmtf_ref[...]

    def one(x, a_ref, c_ref, s_ref, xp_ref, xm_ref):
        yr = _dot(x, dr)
        yi = _dot(x, di)
        fr = _dot(drt, yr) - _dot(dit, yi)
        fi = _dot(drt, yi) + _dot(dit, yr)
        m2 = fr * fr + fi * fi
        mag = jnp.sqrt(m2)
        amp = jnp.log1p(mag)
        rinv = 1.0 / jnp.maximum(mag, 1e-30)
        cosp = fr * rinv
        sinp = fi * rinv
        a_ref[0] = _dot(_dot(pma, amp), pmt) + _dot(_dot(pmb, amp), pmtf)
        c_ref[0] = _dot(_dot(pma, cosp), pmt) + _dot(_dot(pmb, cosp), pmtf)
        s_ref[0] = _dot(_dot(pma, sinp), pmt) - _dot(_dot(pmb, sinp), pmtf)
        xp_ref[0] = _dot(_dot(pm, x), pmt)
        xm_ref[0] = jnp.mean(x, axis=1, keepdims=True).mean(axis=0,
                                                           keepdims=True)

    one(xv_ref[0], av_ref, cv_ref, sv_ref, pv_ref, mv_ref)
    one(xi_ref[0], ai_ref, ci_ref, si_ref, pi_ref, mi_ref)


def _spectral(vis_s, ir_s, dr, di, drt, dit, pm, pmt, pma, pmb, pmtf):
    nimg = vis_s.shape[0]
    pool_s = jax.ShapeDtypeStruct((nimg, HP, WP), F32)
    mean_s = jax.ShapeDtypeStruct((nimg, 1, 1), F32)
    small = pl.BlockSpec((1, HP, WP), lambda g: (g, 0, 0))
    tiny = pl.BlockSpec((1, 1, 1), lambda g: (g, 0, 0))
    full = lambda s: pl.BlockSpec(s, lambda g: tuple(0 for _ in s))
    return pl.pallas_call(
        _spectral_body,
        grid=(nimg,),
        in_specs=[
            pl.BlockSpec((1, H, W), lambda g: (g, 0, 0)),
            pl.BlockSpec((1, H, W), lambda g: (g, 0, 0)),
            full((H, W)), full((H, W)),
            full((FTOP, W)), full((FTOP, W)),
            full((HP, W)), full((W, HP)),
            full((HP, FTOP)), full((HP, FTOP)), full((W, HP)),
        ],
        out_specs=(small, small, small, small, small, small, small, small,
                   tiny, tiny),
        out_shape=(pool_s,) * 8 + (mean_s,) * 2,
    )(vis_s, ir_s, dr, di, drt, dit, pm, pmt, pma, pmb, pmtf)


def _intent_body(pooled_ref, rw1t_ref, rb1_ref, rw2t_ref, rb2_ref,
                 iat_ref, iab_ref, ipt_ref, ipb_ref, a_ref, p_ref):
    pooled = pooled_ref[...]
    h = _gelu(_dot(pooled, rw1t_ref[...]) + rb1_ref[...])
    lg = _dot(h, rw2t_ref[...]) + rb2_ref[...]
    m = jnp.max(lg, axis=-1, keepdims=True)
    e = jnp.exp(lg - m)
    intent = e / jnp.sum(e, axis=-1, keepdims=True)
    a_ref[...] = _dot(intent, iat_ref[...]) + iab_ref[...]
    p_ref[...] = _dot(intent, ipt_ref[...]) + ipb_ref[...]


def _intent(pooled, p):
    rw1t = p['rW1'].T
    rw2t = p['rW2'].T
    iat = p['iaW'][:, :NPROMPT].T
    ipt = p['ipW'][:, :NPROMPT].T
    full = lambda s: pl.BlockSpec(s, lambda: tuple(0 for _ in s))
    return pl.pallas_call(
        _intent_body,
        in_specs=[full((B, 2 * C)), full((2 * C, 128)), full((1, 128)),
                  full((128, NPROMPT)), full((1, NPROMPT)),
                  full((NPROMPT, IDIM)), full((1, IDIM)),
                  full((NPROMPT, IDIM)), full((1, IDIM))],
        out_specs=(full((B, IDIM)), full((B, IDIM))),
        out_shape=(jax.ShapeDtypeStruct((B, IDIM), F32),
                   jax.ShapeDtypeStruct((B, IDIM), F32)),
    )(pooled, rw1t, p['rb1'][None], rw2t, p['rb2'][None],
      iat, p['iab'][None], ipt, p['ipb'][None])


def _score_body(x_ref, w1_ref, b1_ref, w2_ref, b2_ref, w3_ref, b3_ref, o_ref):
    h = _gelu(_dot(x_ref[...], w1_ref[...]) + b1_ref[...])
    h = _gelu(_dot(h, w2_ref[...]) + b2_ref[...])
    o_ref[...] = _dot(h, w3_ref[...]) + b3_ref[...]


def _scores(feat, w1, b1, w2, b2, w3, b3):
    rows, fdim = feat.shape
    blk = 784
    grid = rows // blk
    full = lambda s: pl.BlockSpec(s, lambda g: tuple(0 for _ in s))
    out = pl.pallas_call(
        _score_body,
        grid=(grid,),
        in_specs=[pl.BlockSpec((blk, fdim), lambda g: (g, 0)),
                  full((fdim, HID)), full((1, HID)),
                  full((HID, HID)), full((1, HID)),
                  full((HID, 1)), full((1, 1))],
        out_specs=pl.BlockSpec((blk, 1), lambda g: (g, 0)),
        out_shape=jax.ShapeDtypeStruct((rows, 1), F32),
    )(feat, w1.T, b1[None], w2.T, b2[None], w3.T, b3[None])
    return out


def _select_body(k, srow_ref, scol_ref, rep_ref, ctx_ref, selc_ref, selr_ref):
    srow = srow_ref[0]
    nchunk = 8
    cb = N // nchunk
    for ib in range(nchunk):
        si = scol_ref[0, ib * cb:(ib + 1) * cb, :]
        gt = jnp.sum((srow > si).astype(F32), axis=1, keepdims=True)
        jm = jax.lax.broadcasted_iota(jnp.int32, (cb, N), 1)
        im = jax.lax.broadcasted_iota(jnp.int32, (cb, N), 0) + ib * cb
        eq = jnp.sum(((srow == si) & (jm < im)).astype(F32),
                     axis=1, keepdims=True)
        sel = ((gt + eq) < k).astype(F32)
        selc_ref[ib * cb:(ib + 1) * cb, :] = sel
        selr_ref[:, ib * cb:(ib + 1) * cb] = sel.T
    selr = selr_ref[...]
    for ib in range(nchunk):
        jm = jax.lax.broadcasted_iota(jnp.int32, (cb, N), 1)
        im = jax.lax.broadcasted_iota(jnp.int32, (cb, N), 0) + ib * cb
        pos = jnp.sum(selr * (jm < im).astype(F32), axis=1, keepdims=True)
        sel = selc_ref[ib * cb:(ib + 1) * cb, :]
        ri = jax.lax.broadcasted_iota(jnp.int32, (cb, k), 1)
        oh = ((pos.astype(jnp.int32) == ri) & (sel > 0.5)).astype(F32)
        contrib = _dg(oh, rep_ref[0, ib * cb:(ib + 1) * cb, :],
                      ((0,), (0,)))
        if ib == 0:
            ctx_ref[0] = contrib
        else:
            ctx_ref[0] += contrib


def _select_gather(scores, rep, k):
    b, n, d = rep.shape
    srow = scores.reshape(b, 1, n)
    scol = scores.reshape(b, n, 1)
    return pl.pallas_call(
        functools.partial(_select_body, k),
        grid=(b,),
        in_specs=[pl.BlockSpec((1, 1, n), lambda g: (g, 0, 0)),
                  pl.BlockSpec((1, n, 1), lambda g: (g, 0, 0)),
                  pl.BlockSpec((1, n, d), lambda g: (g, 0, 0))],
        out_specs=pl.BlockSpec((1, k, d), lambda g: (g, 0, 0)),
        out_shape=jax.ShapeDtypeStruct((b, k, d), F32),
        scratch_shapes=[pltpu.VMEM((n, 1), F32),
                        pltpu.VMEM((1, n), F32)],
    )(srow, scol, rep)


def _attn_body(d, q_ref, ctx_ref, n1g_ref, n1b_ref, wqt_ref, bq_ref,
               wkt_ref, bk_ref, wvt_ref, bv_ref, wot_ref, bo_ref,
               n2g_ref, n2b_ref, w1t_ref, b1_ref, w2t_ref, b2_ref, o_ref):
    hd = d // HEADS
    scale = hd ** -0.5
    qin = q_ref[0]
    m = jnp.mean(qin, axis=-1, keepdims=True)
    v = jnp.mean((qin - m) ** 2, axis=-1, keepdims=True)
    qn = (qin - m) / jnp.sqrt(v + 1e-5) * n1g_ref[...] + n1b_ref[...]
    q = _dot(qn, wqt_ref[...]) + bq_ref[...]
    c = ctx_ref[0]
    kk = _dot(c, wkt_ref[...]) + bk_ref[...]
    vv = _dot(c, wvt_ref[...]) + bv_ref[...]
    outs = []
    for h in range(HEADS):
        sl = slice(h * hd, (h + 1) * hd)
        lg = _dg(q[:, sl], kk[:, sl], ((1,), (1,))) * scale
        mx = jnp.max(lg, axis=-1, keepdims=True)
        e = jnp.exp(lg - mx)
        a = e / jnp.sum(e, axis=-1, keepdims=True)
        outs.append(_dot(a, vv[:, sl]))
    o = jnp.concatenate(outs, axis=1)
    o = qin + _dot(o, wot_ref[...]) + bo_ref[...]
    m2 = jnp.mean(o, axis=-1, keepdims=True)
    v2 = jnp.mean((o - m2) ** 2, axis=-1, keepdims=True)
    o2 = (o - m2) / jnp.sqrt(v2 + 1e-5) * n2g_ref[...] + n2b_ref[...]
    hm = _gelu(_dot(o2, w1t_ref[...]) + b1_ref[...])
    o_ref[0] = o + _dot(hm, w2t_ref[...]) + b2_ref[...]


def _cross_attn(q_in, ctx, p, pre):
    b, n, d = q_in.shape
    k = ctx.shape[1]
    qb = 392
    grid = (b, n // qb)
    wfull = lambda s: pl.BlockSpec(s, lambda gb, gi: tuple(0 for _ in s))
    args = [q_in, ctx,
            p[pre + 'n1g'][None], p[pre + 'n1b'][None],
            p[pre + 'Wq'].T, p[pre + 'bq'][None],
            p[pre + 'Wk'].T, p[pre + 'bk'][None],
            p[pre + 'Wv'].T, p[pre + 'bv'][None],
            p[pre + 'Wo'].T, p[pre + 'bo'][None],
            p[pre + 'n2g'][None], p[pre + 'n2b'][None],
            p[pre + 'W1'].T, p[pre + 'b1'][None],
            p[pre + 'W2'].T, p[pre + 'b2'][None]]
    in_specs = [pl.BlockSpec((1, qb, d), lambda gb, gi: (gb, gi, 0)),
                pl.BlockSpec((1, k, d), lambda gb, gi: (gb, 0, 0)),
                wfull((1, d)), wfull((1, d)),
                wfull((d, d)), wfull((1, d)),
                wfull((d, d)), wfull((1, d)),
                wfull((d, d)), wfull((1, d)),
                wfull((d, d)), wfull((1, d)),
                wfull((1, d)), wfull((1, d)),
                wfull((d, 2 * d)), wfull((1, 2 * d)),
                wfull((2 * d, d)), wfull((1, d))]
    return pl.pallas_call(
        functools.partial(_attn_body, d),
        grid=grid,
        in_specs=in_specs,
        out_specs=pl.BlockSpec((1, qb, d), lambda gb, gi: (gb, gi, 0)),
        out_shape=jax.ShapeDtypeStruct((b, n, d), F32),
    )(*args)


def _upbase_body(f_ref, v_ref, i_ref, e_ref, et_ref, o_ref):
    up = _dot(e_ref[...], _dot(f_ref[0], et_ref[...]))
    o_ref[0] = 0.5 * (v_ref[0] + i_ref[0]) + up


def _upbase(fmap, vis, ir, e, et):
    nimg = fmap.shape[0]
    return pl.pallas_call(
        _upbase_body,
        grid=(nimg,),
        in_specs=[pl.BlockSpec((1, HP, WP), lambda g: (g, 0, 0)),
                  pl.BlockSpec((1, H, W), lambda g: (g, 0, 0)),
                  pl.BlockSpec((1, H, W), lambda g: (g, 0, 0)),
                  pl.BlockSpec((H, HP), lambda g: (0, 0)),
                  pl.BlockSpec((HP, W), lambda g: (0, 0))],
        out_specs=pl.BlockSpec((1, H, W), lambda g: (g, 0, 0)),
        out_shape=jax.ShapeDtypeStruct((nimg, H, W), F32),
    )(fmap, vis, ir, e, et)


def _chan_mm_body(act, w_ref, b_ref, x_ref, o_ref):
    r = _dg(w_ref[...], x_ref[0], ((1,), (0,))) + b_ref[...]
    o_ref[0] = _gelu(r) if act else r


def _chan_matmul(x, wmat, bias, act):
    b, cin, cols = x.shape
    cout = wmat.shape[0]
    blk = cols // 8
    return pl.pallas_call(
        functools.partial(_chan_mm_body, act),
        grid=(b, 8),
        in_specs=[pl.BlockSpec((cout, cin), lambda gb, gi: (0, 0)),
                  pl.BlockSpec((cout, 1), lambda gb, gi: (0, 0)),
                  pl.BlockSpec((1, cin, blk), lambda gb, gi: (gb, 0, gi))],
        out_specs=pl.BlockSpec((1, cout, blk), lambda gb, gi: (gb, 0, gi)),
        out_shape=jax.ShapeDtypeStruct((b, cout, cols), F32),
    )(wmat, bias[:, None], x)


_STRIP = 28 * W


def _conv3_body(w_ref, b_ref, m0_ref, m2_ref, e0_ref, e2_ref,
                xp_ref, xc_ref, xn_ref, o_ref):
    i = pl.program_id(1)
    w = w_ref[...]
    xx = jnp.concatenate([xp_ref[0], xc_ref[0], xn_ref[0]], axis=1)
    first = (i == 0).astype(F32)
    last = (i == pl.num_programs(1) - 1).astype(F32)
    acc = None
    for dy in range(3):
        tmp = None
        for dx in range(3):
            s = _STRIP + (dy - 1) * W + (dx - 1)
            xs = xx[:, s:s + _STRIP]
            g = dy * 3 + dx
            t = _dg(w[:, g * C:(g + 1) * C], xs, ((1,), (0,)))
            if dx == 0:
                t = t * m0_ref[...]
            elif dx == 2:
                t = t * m2_ref[...]
            tmp = t if tmp is None else tmp + t
        if dy == 0:
            tmp = tmp * (1.0 - first * e0_ref[...])
        elif dy == 2:
            tmp = tmp * (1.0 - last * e2_ref[...])
        acc = tmp if acc is None else acc + tmp
    o_ref[0] = acc + b_ref[...]


def _conv3(h1_flat, w2, bias, m0, m2, e0, e2):
    full = lambda s: pl.BlockSpec(s, lambda gb, gi: tuple(0 for _ in s))
    return pl.pallas_call(
        _conv3_body,
        grid=(B, 8),
        in_specs=[full((C, 9 * C)), full((C, 1)),
                  full((1, _STRIP)), full((1, _STRIP)),
                  full((1, _STRIP)), full((1, _STRIP)),
                  pl.BlockSpec((1, C, _STRIP),
                               lambda gb, gi: (gb, 0, jnp.maximum(gi - 1, 0))),
                  pl.BlockSpec((1, C, _STRIP),
                               lambda gb, gi: (gb, 0, gi)),
                  pl.BlockSpec((1, C, _STRIP),
                               lambda gb, gi: (gb, 0, jnp.minimum(gi + 1, 7)))],
        out_specs=pl.BlockSpec((1, C, _STRIP), lambda gb, gi: (gb, 0, gi)),
        out_shape=jax.ShapeDtypeStruct((B, C, H * W), F32),
    )(w2, bias[:, None], m0, m2, e0, e2, h1_flat, h1_flat, h1_flat)


def kernel(vis, ir, params):
    p = params
    dr, di = _dft_mats()
    pm, pmt = _pool_mats()
    pma, pmb, pmtf = _sym_pool_mats()
    e, et = _expand_mats()
    drt = jnp.asarray(dr[:FTOP].copy()); dit = jnp.asarray(di[:FTOP].copy())
    dr = jnp.asarray(dr); di = jnp.asarray(di)
    pm = jnp.asarray(pm); pmt = jnp.asarray(pmt)
    pma = jnp.asarray(pma); pmb = jnp.asarray(pmb); pmtf = jnp.asarray(pmtf)
    e = jnp.asarray(e); et = jnp.asarray(et)
    pos = jnp.asarray(_pos_feat())

    (av_p, ai_p, cv_p, ci_p, sv_p, si_p, pv_p, pi_p, mv, mi) = _spectral(
        vis.reshape(B * C, H, W), ir.reshape(B * C, H, W), dr, di, drt, dit,
        pm, pmt, pma, pmb, pmtf)

    def tok(arr3):
        return arr3.reshape(B, C, N).transpose(0, 2, 1)

    av_t, ai_t = tok(av_p), tok(ai_p)
    cv_t, ci_t = tok(cv_p), tok(ci_p)
    sv_t, si_t = tok(sv_p), tok(si_p)
    vis_t, ir_t = tok(pv_p), tok(pi_p)

    pooled = jnp.concatenate([mv.reshape(B, C), mi.reshape(B, C)], axis=1)
    amp_intent, phase_intent = _intent(pooled, p)

    pos_b = jnp.broadcast_to(pos, (B, N, 2))
    ai_b = jnp.broadcast_to(amp_intent[:, None, :], (B, N, IDIM))
    pi_b = jnp.broadcast_to(phase_intent[:, None, :], (B, N, IDIM))
    amp_feat = jnp.concatenate([av_t, ai_t, vis_t, ir_t, pos_b, ai_b],
                               axis=-1).reshape(B * N, -1)
    ph_feat = jnp.concatenate([cv_t, sv_t, ci_t, si_t, av_t, ai_t,
                               vis_t, ir_t, pos_b, pi_b],
                              axis=-1).reshape(B * N, -1)
    amp_scores = _scores(amp_feat, p['aW1'], p['ab1'], p['aW2'], p['ab2'],
                         p['aW3'], p['ab3']).reshape(B, N)
    ph_scores = _scores(ph_feat, p['pW1'], p['pb1'], p['pW2'], p['pb2'],
                        p['pW3'], p['pb3']).reshape(B, N)

    amp_rep = 0.5 * (av_t + ai_t)
    ph_rep = jnp.concatenate([0.5 * (cv_t + ci_t), 0.5 * (sv_t + si_t)],
                             axis=-1)
    amp_ctx = _select_gather(amp_scores, amp_rep, KSEL)
    ph_ctx = _select_gather(ph_scores, ph_rep, KSEL)

    q_amp = 0.5 * (vis_t + ir_t)
    q_ph = jnp.concatenate([vis_t, ir_t], axis=-1)
    amp_out = _cross_attn(q_amp, amp_ctx, p, 'ac_')
    ph_out = _cross_attn(q_ph, ph_ctx, p, 'pc_')
    ph_red = 0.5 * (ph_out[..., :C] + ph_out[..., C:])
    fused = amp_out + ph_red

    fmap = fused.transpose(0, 2, 1).reshape(B * C, HP, WP)
    base = _upbase(fmap, vis.reshape(B * C, H, W), ir.reshape(B * C, H, W),
                   e, et)
    base = base.reshape(B, C, H * W)
    h1 = _chan_matmul(base, p['c1W'], p['c1b'], act=True)

    w2 = p['c2W'].transpose(0, 2, 3, 1).reshape(C, 9 * C)
    lane = np.arange(_STRIP) % W
    m0 = jnp.asarray((lane != 0).astype(np.float32)[None])
    m2 = jnp.asarray((lane != W - 1).astype(np.float32)[None])
    edge = np.zeros((1, _STRIP), np.float32)
    e0 = edge.copy(); e0[0, :W] = 1.0
    e2 = edge.copy(); e2[0, -W:] = 1.0
    out = _conv3(h1, w2, p['c2b'], m0, m2, jnp.asarray(e0), jnp.asarray(e2))
    return out.reshape(B, C, H, W)
